# baseline (device time: 13722 ns/iter reference)
import jax
import jax.numpy as jnp
from jax import lax
from jax.experimental import pallas as pl
from jax.experimental.pallas import tpu as pltpu

_CHUNKS = ((0, 88), (88, 88), (176, 80))
_ORDERS = ((2, 1, 0), (1, 0, 2), (0, 2, 1))


def kernel(x):
    m, n = x.shape[-2], x.shape[-1]

    def body(x_ref, out_ref, acc_ref, recv_buf, send_sems, recv_sems, out_sems):
        my_x = lax.axis_index("x")
        my_y = lax.axis_index("y")
        my_z = lax.axis_index("z")
        nbr_by_axis = [
            (1 - my_x, my_y, my_z),
            (my_x, 1 - my_y, my_z),
            (my_x, my_y, 1 - my_z),
        ]
        x2d = x_ref.at[0, 0, 0]

        barrier_sem = pltpu.get_barrier_semaphore()
        for nbr in nbr_by_axis:
            pl.semaphore_signal(
                barrier_sem, inc=1,
                device_id=nbr, device_id_type=pl.DeviceIdType.MESH,
            )
        pl.semaphore_wait(barrier_sem, 3)

        def exchange(p, c):
            src = x2d if p == 0 else acc_ref
            r0, rs = _CHUNKS[c]
            rdma = pltpu.make_async_remote_copy(
                src_ref=src.at[pl.ds(r0, rs)],
                dst_ref=recv_buf.at[p, pl.ds(r0, rs)],
                send_sem=send_sems.at[p, c],
                recv_sem=recv_sems.at[p, c],
                device_id=nbr_by_axis[_ORDERS[c][p]],
                device_id_type=pl.DeviceIdType.MESH,
            )
            rdma.start()
            return rdma

        out_dmas = [None, None, None]
        rdmas = [exchange(0, c) for c in range(3)]
        for p in range(3):
            for c, (r0, rs) in enumerate(_CHUNKS):
                rdmas[c].wait()
                if p == 0:
                    acc_ref[pl.ds(r0, rs), :] = (
                        x2d[pl.ds(r0, rs), :] + recv_buf[0, pl.ds(r0, rs), :]
                    )
                else:
                    acc_ref[pl.ds(r0, rs), :] += recv_buf[p, pl.ds(r0, rs), :]
                if p < 2:
                    rdmas[c] = exchange(p + 1, c)
                else:
                    dma = pltpu.make_async_copy(
                        acc_ref.at[pl.ds(r0, rs)],
                        out_ref.at[pl.ds(r0, rs)],
                        out_sems.at[c],
                    )
                    dma.start()
                    out_dmas[c] = dma
        for dma in out_dmas:
            dma.wait()

    return pl.pallas_call(
        body,
        out_shape=jax.ShapeDtypeStruct((m, n), x.dtype),
        in_specs=[pl.BlockSpec(memory_space=pltpu.VMEM)],
        out_specs=pl.BlockSpec(memory_space=pl.ANY),
        scratch_shapes=[
            pltpu.VMEM((m, n), x.dtype),
            pltpu.VMEM((3, m, n), x.dtype),
            pltpu.SemaphoreType.DMA((3, 3)),
            pltpu.SemaphoreType.DMA((3, 3)),
            pltpu.SemaphoreType.DMA((3,)),
        ],
        compiler_params=pltpu.CompilerParams(collective_id=0),
    )(x)


# device time: 13332 ns/iter; 1.0293x vs baseline; 1.0293x over previous
import jax
from jax import lax
from jax.experimental import pallas as pl
from jax.experimental.pallas import tpu as pltpu

_CHUNKS = ((0, 88), (88, 88), (176, 80))
_ORDERS = ((2, 1, 0), (1, 0, 2), (0, 2, 1))


def kernel(x):
    m, n = x.shape[-2], x.shape[-1]

    def body(x_ref, out_ref, recv_buf, send_sems, recv_sems):
        my_x = lax.axis_index("x")
        my_y = lax.axis_index("y")
        my_z = lax.axis_index("z")
        nbr_by_axis = [
            (1 - my_x, my_y, my_z),
            (my_x, 1 - my_y, my_z),
            (my_x, my_y, 1 - my_z),
        ]
        x2d = x_ref.at[0, 0, 0]

        barrier_sem = pltpu.get_barrier_semaphore()
        for nbr in nbr_by_axis:
            pl.semaphore_signal(
                barrier_sem, inc=1,
                device_id=nbr, device_id_type=pl.DeviceIdType.MESH,
            )
        pl.semaphore_wait(barrier_sem, 3)

        def exchange(p, c):
            src = x2d if p == 0 else out_ref
            r0, rs = _CHUNKS[c]
            rdma = pltpu.make_async_remote_copy(
                src_ref=src.at[pl.ds(r0, rs)],
                dst_ref=recv_buf.at[p, pl.ds(r0, rs)],
                send_sem=send_sems.at[p, c],
                recv_sem=recv_sems.at[p, c],
                device_id=nbr_by_axis[_ORDERS[c][p]],
                device_id_type=pl.DeviceIdType.MESH,
            )
            rdma.start()
            return rdma

        rdmas = [exchange(0, c) for c in range(3)]
        for p in range(3):
            for c, (r0, rs) in enumerate(_CHUNKS):
                rdmas[c].wait()
                if p == 0:
                    out_ref[pl.ds(r0, rs), :] = (
                        x2d[pl.ds(r0, rs), :] + recv_buf[0, pl.ds(r0, rs), :]
                    )
                else:
                    out_ref[pl.ds(r0, rs), :] += recv_buf[p, pl.ds(r0, rs), :]
                if p < 2:
                    rdmas[c] = exchange(p + 1, c)

    return pl.pallas_call(
        body,
        out_shape=jax.ShapeDtypeStruct((m, n), x.dtype),
        in_specs=[pl.BlockSpec(memory_space=pltpu.VMEM)],
        out_specs=pl.BlockSpec(memory_space=pltpu.VMEM),
        scratch_shapes=[
            pltpu.VMEM((3, m, n), x.dtype),
            pltpu.SemaphoreType.DMA((3, 3)),
            pltpu.SemaphoreType.DMA((3, 3)),
        ],
        compiler_params=pltpu.CompilerParams(collective_id=0),
    )(x)
